# baseline (speedup 1.0000x reference)
import jax
import jax.numpy as jnp
from jax import lax
from jax.experimental import pallas as pl
from jax.experimental.pallas import tpu as pltpu
from jax.experimental.pallas import tpu_sc as plsc

N = 10000
D = 256
H = 512
E = 160000
B = 16384
NU = 4000

NP = 10240
MB = 256
GB = 1024
NB = B // GB
NT = 16
KB = 32
NBK = 320
EPT = NBK * KB
EPAD = NT * EPT
ROWS_PT = NP // NT
CH = 128
NCH = H // CH


def _mm_body(x_ref, w_ref, o_ref):
    res = jnp.dot(x_ref[...], w_ref[...], preferred_element_type=jnp.float32)
    for c in range(NCH):
        o_ref[c] = res[:, c * CH:(c + 1) * CH]


def _mm(x, w):
    k = x.shape[1]
    return pl.pallas_call(
        _mm_body,
        grid=(NP // MB,),
        in_specs=[pl.BlockSpec((MB, k), lambda m: (m, 0)),
                  pl.BlockSpec((k, H), lambda m: (0, 0))],
        out_specs=pl.BlockSpec((NCH, MB, CH), lambda m: (0, m, 0)),
        out_shape=jax.ShapeDtypeStruct((NCH, NP, CH), jnp.float32),
    )(x, w)


def _spmm_sc_body(sup_ref, pk_ref, ew_ref, zer_ref, out_ref, *rest):
    pbuf, ewbuf = rest[0], rest[1]
    gixs = rest[2:6]
    dstbs = rest[6:10]
    rows = rest[10:14]
    acc = rest[14]
    gsems = rest[15:19]
    ssems = rest[19:23]
    cid = lax.axis_index("c")
    tid = lax.axis_index("s")
    G0 = (0, 1)
    G1 = (2, 3)
    NSUP = NBK // 4

    pltpu.sync_copy(pk_ref.at[tid], pbuf)
    pltpu.sync_copy(ew_ref.at[tid], ewbuf)

    def drain_scat(i):
        pltpu.make_async_copy(sup_ref.at[pl.ds(0, KB)], rows[i],
                              ssems[i]).wait()

    def fire(i, b, off):
        def gi(g, _):
            v = pbuf[pl.ds(b * KB + g * 16, 16)]
            gixs[i][pl.ds(g * 16, 16)] = (
                lax.shift_right_logical(v, 14) + off)
            dstbs[i][0, pl.ds(g * 16, 16)] = lax.bitwise_and(v, 16383)
            return 0
        lax.fori_loop(0, KB // 16, gi, 0)
        pltpu.async_copy(sup_ref.at[gixs[i]], rows[i], gsems[i])

    def consume(i, b):
        pltpu.make_async_copy(sup_ref.at[pl.ds(0, KB)], rows[i],
                              gsems[i]).wait()

        def mg(g, _):
            ewv = ewbuf[pl.ds(b * KB + g * 16, 16)]
            for j in range(16):
                w = ewv.at[jnp.full((16,), j, jnp.int32)].get(
                    mode='promise_in_bounds')
                r = g * 16 + j
                for f in range(CH // 16):
                    rows[i][r, pl.ds(f * 16, 16)] = (
                        rows[i][r, pl.ds(f * 16, 16)] * w)
            return 0
        lax.fori_loop(0, KB // 16, mg, 0)
        pltpu.async_copy(rows[i], acc.at[dstbs[i].at[0]], ssems[i], add=True)

    def chunk_pass(c_local, _):
        c = cid * (NCH // 2) + c_local
        off = c * NP
        pltpu.sync_copy(zer_ref, acc.at[pl.ds(tid * ROWS_PT, ROWS_PT)])
        plsc.subcore_barrier()

        fire(0, 0, off)
        fire(1, 1, off)

        def super_iter(t, _):
            @pl.when(t > 0)
            def _():
                drain_scat(2)
                drain_scat(3)
            fire(2, 4 * t + 2, off)
            fire(3, 4 * t + 3, off)
            consume(0, 4 * t)
            consume(1, 4 * t + 1)

            @pl.when(t < NSUP - 1)
            def _():
                drain_scat(0)
                drain_scat(1)
                fire(0, 4 * t + 4, off)
                fire(1, 4 * t + 5, off)
            consume(2, 4 * t + 2)
            consume(3, 4 * t + 3)
            return 0
        lax.fori_loop(0, NSUP, super_iter, 0)
        for i in range(4):
            drain_scat(i)
        plsc.subcore_barrier()
        pltpu.sync_copy(acc.at[pl.ds(tid * ROWS_PT, ROWS_PT)],
                        out_ref.at[c, pl.ds(tid * ROWS_PT, ROWS_PT)])
        return 0
    lax.fori_loop(0, NCH // 2, chunk_pass, 0)


_SPMM_SC = pl.kernel(
    _spmm_sc_body,
    out_type=jax.ShapeDtypeStruct((NCH, NP, CH), jnp.float32),
    mesh=plsc.VectorSubcoreMesh(core_axis_name="c", subcore_axis_name="s"),
    scratch_types=(
        [pltpu.VMEM((NBK * KB,), jnp.int32),
         pltpu.VMEM((NBK * KB,), jnp.float32)]
        + [pltpu.VMEM((KB,), jnp.int32) for _ in range(4)]
        + [pltpu.VMEM((1, KB), jnp.int32) for _ in range(4)]
        + [pltpu.VMEM((KB, CH), jnp.float32) for _ in range(4)]
        + [pltpu.VMEM_SHARED((NP, CH), jnp.float32)]
        + [pltpu.SemaphoreType.DMA for _ in range(8)]
    ),
)


def _spmm(sup4, packed, ew_t, zer):
    return _SPMM_SC(sup4.reshape(NCH * NP, CH), packed.reshape(NT, EPT),
                    ew_t.reshape(NT, EPT), zer)


def _mm2_body(agg_ref, b_ref, w_ref, x_ref, o_ref):
    w = w_ref[...]
    acc = jnp.zeros((MB, H), jnp.float32)
    for c in range(NCH):
        xc = jax.nn.relu(agg_ref[c] + b_ref[0:1, c * CH:(c + 1) * CH])
        x_ref[c] = xc
        acc += jnp.dot(xc, w[c * CH:(c + 1) * CH, :],
                       preferred_element_type=jnp.float32)
    for c in range(NCH):
        o_ref[c] = acc[:, c * CH:(c + 1) * CH]


def _mm2(agg, b, w):
    return pl.pallas_call(
        _mm2_body,
        grid=(NP // MB,),
        in_specs=[pl.BlockSpec((NCH, MB, CH), lambda m: (0, m, 0)),
                  pl.BlockSpec((1, H), lambda m: (0, 0)),
                  pl.BlockSpec((H, H), lambda m: (0, 0))],
        out_specs=[pl.BlockSpec((NCH, MB, CH), lambda m: (0, m, 0)),
                   pl.BlockSpec((NCH, MB, CH), lambda m: (0, m, 0))],
        out_shape=[jax.ShapeDtypeStruct((NCH, NP, CH), jnp.float32),
                   jax.ShapeDtypeStruct((NCH, NP, CH), jnp.float32)],
    )(agg, b, w)


def _score_body(x1_ref, agg2_ref, b_ref, a1_ref, a2_ref, o_ref):
    acc = jnp.zeros((MB, 128), jnp.float32)
    for c in range(NCH):
        rs = slice(c * CH, (c + 1) * CH)
        acc += jnp.dot(x1_ref[c], a1_ref[rs, :],
                       preferred_element_type=jnp.float32)
        x2c = jax.nn.relu(agg2_ref[c] + b_ref[0:1, rs])
        acc += jnp.dot(x2c, a2_ref[rs, :], preferred_element_type=jnp.float32)
    o_ref[...] = acc


def _score(x1, agg2, b, a1p, a2p):
    return pl.pallas_call(
        _score_body,
        grid=(NP // MB,),
        in_specs=[pl.BlockSpec((NCH, MB, CH), lambda m: (0, m, 0)),
                  pl.BlockSpec((NCH, MB, CH), lambda m: (0, m, 0)),
                  pl.BlockSpec((1, H), lambda m: (0, 0)),
                  pl.BlockSpec((H, 128), lambda m: (0, 0)),
                  pl.BlockSpec((H, 128), lambda m: (0, 0))],
        out_specs=pl.BlockSpec((MB, 128), lambda m: (m, 0)),
        out_shape=jax.ShapeDtypeStruct((NP, 128), jnp.float32),
    )(x1, agg2, b, a1p, a2p)


def _gather_body(u_ref, i_ref, sc_ref, gu_ref, gi_ref):
    def body(k, _):
        u = u_ref[0, 0, k]
        i = i_ref[0, 0, k] + NU
        gu_ref[pl.ds(k, 1), :] = sc_ref[pl.ds(u, 1), :]
        gi_ref[pl.ds(k, 1), :] = sc_ref[pl.ds(i, 1), :]
        return 0

    lax.fori_loop(0, GB, body, 0)


def _gather(u3, i3, scores):
    return pl.pallas_call(
        _gather_body,
        grid=(NB,),
        in_specs=[
            pl.BlockSpec((1, 1, GB), lambda b: (b, 0, 0),
                         memory_space=pltpu.SMEM),
            pl.BlockSpec((1, 1, GB), lambda b: (b, 0, 0),
                         memory_space=pltpu.SMEM),
            pl.BlockSpec((NP, 128), lambda b: (0, 0)),
        ],
        out_specs=[pl.BlockSpec((GB, 128), lambda b: (b, 0)),
                   pl.BlockSpec((GB, 128), lambda b: (b, 0))],
        out_shape=[jax.ShapeDtypeStruct((B, 128), jnp.float32),
                   jax.ShapeDtypeStruct((B, 128), jnp.float32)],
    )(u3, i3, scores)


def _bce_body(gu_ref, gi_ref, y_ref, c_ref, o_ref):
    @pl.when(pl.program_id(0) == 0)
    def _():
        o_ref[...] = jnp.zeros_like(o_ref)

    gu = gu_ref[...]
    gi = gi_ref[...]
    lane = lax.broadcasted_iota(jnp.int32, (GB, 128), 1)
    su = jnp.sum(jnp.where(lane == 0, gu, 0.0), axis=1, keepdims=True)
    si = jnp.sum(jnp.where(lane == 1, gi, 0.0), axis=1, keepdims=True)
    du = jnp.sum(jnp.where(lane == 2, gu, 0.0), axis=1, keepdims=True)
    di = jnp.sum(jnp.where(lane == 3, gi, 0.0), axis=1, keepdims=True)
    y = y_ref[...]
    lb = c_ref[0:1, 0:1]
    dub = c_ref[1:2, 0:1]
    dib = c_ref[2:3, 0:1]
    p = jnp.clip(jax.nn.sigmoid(su + si + lb), 1e-7, 1.0 - 1e-7)
    s0 = jnp.sum(y * jnp.log(p) + (1.0 - y) * jnp.log(1.0 - p))
    pu = jnp.clip(jax.nn.sigmoid(du + dub), 1e-7, 1.0 - 1e-7)
    pi = jnp.clip(jax.nn.sigmoid(di + dib), 1e-7, 1.0 - 1e-7)
    s1 = jnp.sum(jnp.log(pu))
    s2 = jnp.sum(jnp.log(1.0 - pu))
    s3 = jnp.sum(jnp.log(pi))
    s4 = jnp.sum(jnp.log(1.0 - pi))
    row = lax.broadcasted_iota(jnp.int32, (8, 128), 0)
    part = (s0 * (row == 0) + s1 * (row == 1) + s2 * (row == 2)
            + s3 * (row == 3) + s4 * (row == 4)).astype(jnp.float32)
    o_ref[...] += part


def _bce(gu, gi, y2, consts):
    return pl.pallas_call(
        _bce_body,
        grid=(NB,),
        in_specs=[pl.BlockSpec((GB, 128), lambda b: (b, 0)),
                  pl.BlockSpec((GB, 128), lambda b: (b, 0)),
                  pl.BlockSpec((GB, 1), lambda b: (b, 0)),
                  pl.BlockSpec((8, 1), lambda b: (0, 0))],
        out_specs=pl.BlockSpec((8, 128), lambda b: (0, 0)),
        out_shape=jax.ShapeDtypeStruct((8, 128), jnp.float32),
    )(gu, gi, y2, consts)


def kernel(V_d1, V_d2, edge_index_s, edge_index_t, edge_weight_s,
           edge_weight_t, W1, b1, W2, b2, lw, lb, duw, dub, diw, dib,
           train_data_s, train_data_t):
    b1r = b1.reshape(1, H)
    b2r = b2.reshape(1, H)
    a1 = jnp.stack([lw[0:H, 0], lw[2 * H:3 * H, 0], duw[0:H, 0],
                    diw[0:H, 0]], axis=1)
    a2 = jnp.stack([lw[H:2 * H, 0], lw[3 * H:4 * H, 0], duw[H:2 * H, 0],
                    diw[H:2 * H, 0]], axis=1)
    a1p = jnp.pad(a1, ((0, 0), (0, 124)))
    a2p = jnp.pad(a2, ((0, 0), (0, 124)))
    consts = jnp.pad(jnp.stack([lb, dub, dib], axis=0),
                     ((0, 5), (0, 0)))
    zer = jnp.zeros((ROWS_PT, CH), jnp.float32)

    sums = []
    for V, ei, ew, td in ((V_d1, edge_index_s, edge_weight_s, train_data_s),
                          (V_d2, edge_index_t, edge_weight_t, train_data_t)):
        vp = jnp.pad(V, ((0, NP - N), (0, 0)))
        src_t = jnp.pad(ei[0], (0, EPAD - E)).reshape(NT, NBK, KB)
        dst_t = jnp.pad(ei[1], (0, EPAD - E)).reshape(NT, NBK, KB)
        ew_t = jnp.pad(ew, (0, EPAD - E)).reshape(NT, NBK, KB)
        packed = src_t * 16384 + dst_t
        u3 = td[:, 0].reshape(NB, 1, GB)
        i3 = td[:, 1].reshape(NB, 1, GB)
        y2 = td[:, 2].astype(jnp.float32).reshape(B, 1)

        sup1 = _mm(vp, W1)
        agg1 = _spmm(sup1, packed, ew_t, zer)
        x1, sup2 = _mm2(agg1, b1r, W2)
        agg2 = _spmm(sup2, packed, ew_t, zer)
        scores = _score(x1, agg2, b2r, a1p, a2p)
        gu, gi = _gather(u3, i3, scores)
        sums.append(_bce(gu, gi, y2, consts))

    s_s, s_t = sums[0][:, 0], sums[1][:, 0]
    loss = -(s_s[0] + s_t[0]) / B
    u_bce = -(s_s[2] + s_t[1]) / (2 * B)
    i_bce = -(s_s[4] + s_t[3]) / (2 * B)
    return loss + (i_bce + u_bce) * 0.1


# speedup vs baseline: 1.0327x; 1.0327x over previous
import jax
import jax.numpy as jnp
from jax import lax
from jax.experimental import pallas as pl
from jax.experimental.pallas import tpu as pltpu
from jax.experimental.pallas import tpu_sc as plsc

N = 10000
D = 256
H = 512
E = 160000
B = 16384
NU = 4000

NP = 10240
MB = 256
GB = 1024
NB = B // GB
NT = 16
KB = 32
NBK = 320
EPT = NBK * KB
EPAD = NT * EPT
ROWS_PT = NP // NT
CH = 128
NCH = H // CH


def _mm_body(x_ref, w_ref, o_ref):
    res = jnp.dot(x_ref[...], w_ref[...], preferred_element_type=jnp.float32)
    for c in range(NCH):
        o_ref[c] = res[:, c * CH:(c + 1) * CH]


def _mm(x, w):
    k = x.shape[1]
    return pl.pallas_call(
        _mm_body,
        grid=(NP // MB,),
        in_specs=[pl.BlockSpec((MB, k), lambda m: (m, 0)),
                  pl.BlockSpec((k, H), lambda m: (0, 0))],
        out_specs=pl.BlockSpec((NCH, MB, CH), lambda m: (0, m, 0)),
        out_shape=jax.ShapeDtypeStruct((NCH, NP, CH), jnp.float32),
    )(x, w)


def _spmm_sc_body(sup_ref, edata_ref, zer_ref, out_ref, *rest):
    ebufs = rest[0:8]
    gixs = rest[8:16]
    rows = rest[16:24]
    acc = rest[24]
    esems = rest[25:33]
    gsems = rest[33:41]
    ssems = rest[41:49]
    cid = lax.axis_index("c")
    tid = lax.axis_index("s")
    G0 = tuple(range(4))
    G1 = tuple(range(4, 8))
    NSUP = NBK // 8

    def drain_scat(i):
        pltpu.make_async_copy(sup_ref.at[pl.ds(0, KB)], rows[i],
                              ssems[i]).wait()

    def fire_quad(grp, qbase, off, guard=None):
        hs = []
        for k, i in enumerate(grp):
            hs.append(pltpu.async_copy(edata_ref.at[tid, qbase + k],
                                       ebufs[i], esems[i]))
        for k, i in enumerate(grp):
            hs[k].wait()
            def gi(g, _, i=i):
                gixs[i][pl.ds(g * 16, 16)] = (
                    ebufs[i][0, pl.ds(g * 16, 16)] + off)
                return 0
            lax.fori_loop(0, KB // 16, gi, 0)
            pltpu.async_copy(sup_ref.at[gixs[i]], rows[i], gsems[i])

    def mul(i):
        def mg(g, _):
            ewv = lax.bitcast_convert_type(ebufs[i][2, pl.ds(g * 16, 16)],
                                           jnp.float32)
            for j in range(16):
                w = ewv.at[jnp.full((16,), j, jnp.int32)].get(
                    mode='promise_in_bounds')
                r = g * 16 + j
                for f in range(CH // 16):
                    rows[i][r, pl.ds(f * 16, 16)] = (
                        rows[i][r, pl.ds(f * 16, 16)] * w)
            return 0
        lax.fori_loop(0, KB // 16, mg, 0)

    def consume_quad(grp):
        for i in grp:
            pltpu.make_async_copy(sup_ref.at[pl.ds(0, KB)], rows[i],
                                  gsems[i]).wait()
            mul(i)
            pltpu.async_copy(rows[i], acc.at[ebufs[i].at[1]], ssems[i],
                             add=True)

    def chunk_pass(c_local, _):
        c = cid * (NCH // 2) + c_local
        off = c * NP
        pltpu.sync_copy(zer_ref, acc.at[pl.ds(tid * ROWS_PT, ROWS_PT)])
        plsc.subcore_barrier()

        fire_quad(G0, 0, off)

        def super_iter(t, _):
            @pl.when(t > 0)
            def _():
                for i in G1:
                    drain_scat(i)
            fire_quad(G1, 8 * t + 4, off)
            consume_quad(G0)

            @pl.when(t < NSUP - 1)
            def _():
                for i in G0:
                    drain_scat(i)
                fire_quad(G0, 8 * t + 8, off)
            consume_quad(G1)
            return 0
        lax.fori_loop(0, NSUP, super_iter, 0)
        for i in range(8):
            drain_scat(i)
        plsc.subcore_barrier()
        pltpu.sync_copy(acc.at[pl.ds(tid * ROWS_PT, ROWS_PT)],
                        out_ref.at[c, pl.ds(tid * ROWS_PT, ROWS_PT)])
        return 0
    lax.fori_loop(0, NCH // 2, chunk_pass, 0)


_SPMM_SC = pl.kernel(
    _spmm_sc_body,
    out_type=jax.ShapeDtypeStruct((NCH, NP, CH), jnp.float32),
    mesh=plsc.VectorSubcoreMesh(core_axis_name="c", subcore_axis_name="s"),
    scratch_types=(
        [pltpu.VMEM((3, KB), jnp.int32) for _ in range(8)]
        + [pltpu.VMEM((KB,), jnp.int32) for _ in range(8)]
        + [pltpu.VMEM((KB, CH), jnp.float32) for _ in range(8)]
        + [pltpu.VMEM_SHARED((NP, CH), jnp.float32)]
        + [pltpu.SemaphoreType.DMA for _ in range(24)]
    ),
)


def _spmm(sup4, edata, zer):
    return _SPMM_SC(sup4.reshape(NCH * NP, CH), edata, zer)


def _mm2_body(agg_ref, b_ref, w_ref, x_ref, o_ref):
    w = w_ref[...]
    acc = jnp.zeros((MB, H), jnp.float32)
    for c in range(NCH):
        xc = jax.nn.relu(agg_ref[c] + b_ref[0:1, c * CH:(c + 1) * CH])
        x_ref[c] = xc
        acc += jnp.dot(xc, w[c * CH:(c + 1) * CH, :],
                       preferred_element_type=jnp.float32)
    for c in range(NCH):
        o_ref[c] = acc[:, c * CH:(c + 1) * CH]


def _mm2(agg, b, w):
    return pl.pallas_call(
        _mm2_body,
        grid=(NP // MB,),
        in_specs=[pl.BlockSpec((NCH, MB, CH), lambda m: (0, m, 0)),
                  pl.BlockSpec((1, H), lambda m: (0, 0)),
                  pl.BlockSpec((H, H), lambda m: (0, 0))],
        out_specs=[pl.BlockSpec((NCH, MB, CH), lambda m: (0, m, 0)),
                   pl.BlockSpec((NCH, MB, CH), lambda m: (0, m, 0))],
        out_shape=[jax.ShapeDtypeStruct((NCH, NP, CH), jnp.float32),
                   jax.ShapeDtypeStruct((NCH, NP, CH), jnp.float32)],
    )(agg, b, w)


def _score_body(x1_ref, agg2_ref, b_ref, a1_ref, a2_ref, o_ref):
    acc = jnp.zeros((MB, 128), jnp.float32)
    for c in range(NCH):
        rs = slice(c * CH, (c + 1) * CH)
        acc += jnp.dot(x1_ref[c], a1_ref[rs, :],
                       preferred_element_type=jnp.float32)
        x2c = jax.nn.relu(agg2_ref[c] + b_ref[0:1, rs])
        acc += jnp.dot(x2c, a2_ref[rs, :], preferred_element_type=jnp.float32)
    o_ref[...] = acc


def _score(x1, agg2, b, a1p, a2p):
    return pl.pallas_call(
        _score_body,
        grid=(NP // MB,),
        in_specs=[pl.BlockSpec((NCH, MB, CH), lambda m: (0, m, 0)),
                  pl.BlockSpec((NCH, MB, CH), lambda m: (0, m, 0)),
                  pl.BlockSpec((1, H), lambda m: (0, 0)),
                  pl.BlockSpec((H, 128), lambda m: (0, 0)),
                  pl.BlockSpec((H, 128), lambda m: (0, 0))],
        out_specs=pl.BlockSpec((MB, 128), lambda m: (m, 0)),
        out_shape=jax.ShapeDtypeStruct((NP, 128), jnp.float32),
    )(x1, agg2, b, a1p, a2p)


def _gather_body(u_ref, i_ref, sc_ref, gu_ref, gi_ref):
    def body(k, _):
        u = u_ref[0, 0, k]
        i = i_ref[0, 0, k] + NU
        gu_ref[pl.ds(k, 1), :] = sc_ref[pl.ds(u, 1), :]
        gi_ref[pl.ds(k, 1), :] = sc_ref[pl.ds(i, 1), :]
        return 0

    lax.fori_loop(0, GB, body, 0)


def _gather(u3, i3, scores):
    return pl.pallas_call(
        _gather_body,
        grid=(NB,),
        in_specs=[
            pl.BlockSpec((1, 1, GB), lambda b: (b, 0, 0),
                         memory_space=pltpu.SMEM),
            pl.BlockSpec((1, 1, GB), lambda b: (b, 0, 0),
                         memory_space=pltpu.SMEM),
            pl.BlockSpec((NP, 128), lambda b: (0, 0)),
        ],
        out_specs=[pl.BlockSpec((GB, 128), lambda b: (b, 0)),
                   pl.BlockSpec((GB, 128), lambda b: (b, 0))],
        out_shape=[jax.ShapeDtypeStruct((B, 128), jnp.float32),
                   jax.ShapeDtypeStruct((B, 128), jnp.float32)],
    )(u3, i3, scores)


def _bce_body(gu_ref, gi_ref, y_ref, c_ref, o_ref):
    @pl.when(pl.program_id(0) == 0)
    def _():
        o_ref[...] = jnp.zeros_like(o_ref)

    gu = gu_ref[...]
    gi = gi_ref[...]
    lane = lax.broadcasted_iota(jnp.int32, (GB, 128), 1)
    su = jnp.sum(jnp.where(lane == 0, gu, 0.0), axis=1, keepdims=True)
    si = jnp.sum(jnp.where(lane == 1, gi, 0.0), axis=1, keepdims=True)
    du = jnp.sum(jnp.where(lane == 2, gu, 0.0), axis=1, keepdims=True)
    di = jnp.sum(jnp.where(lane == 3, gi, 0.0), axis=1, keepdims=True)
    y = y_ref[...]
    lb = c_ref[0:1, 0:1]
    dub = c_ref[1:2, 0:1]
    dib = c_ref[2:3, 0:1]
    p = jnp.clip(jax.nn.sigmoid(su + si + lb), 1e-7, 1.0 - 1e-7)
    s0 = jnp.sum(y * jnp.log(p) + (1.0 - y) * jnp.log(1.0 - p))
    pu = jnp.clip(jax.nn.sigmoid(du + dub), 1e-7, 1.0 - 1e-7)
    pi = jnp.clip(jax.nn.sigmoid(di + dib), 1e-7, 1.0 - 1e-7)
    s1 = jnp.sum(jnp.log(pu))
    s2 = jnp.sum(jnp.log(1.0 - pu))
    s3 = jnp.sum(jnp.log(pi))
    s4 = jnp.sum(jnp.log(1.0 - pi))
    row = lax.broadcasted_iota(jnp.int32, (8, 128), 0)
    part = (s0 * (row == 0) + s1 * (row == 1) + s2 * (row == 2)
            + s3 * (row == 3) + s4 * (row == 4)).astype(jnp.float32)
    o_ref[...] += part


def _bce(gu, gi, y2, consts):
    return pl.pallas_call(
        _bce_body,
        grid=(NB,),
        in_specs=[pl.BlockSpec((GB, 128), lambda b: (b, 0)),
                  pl.BlockSpec((GB, 128), lambda b: (b, 0)),
                  pl.BlockSpec((GB, 1), lambda b: (b, 0)),
                  pl.BlockSpec((8, 1), lambda b: (0, 0))],
        out_specs=pl.BlockSpec((8, 128), lambda b: (0, 0)),
        out_shape=jax.ShapeDtypeStruct((8, 128), jnp.float32),
    )(gu, gi, y2, consts)


def kernel(V_d1, V_d2, edge_index_s, edge_index_t, edge_weight_s,
           edge_weight_t, W1, b1, W2, b2, lw, lb, duw, dub, diw, dib,
           train_data_s, train_data_t):
    b1r = b1.reshape(1, H)
    b2r = b2.reshape(1, H)
    a1 = jnp.stack([lw[0:H, 0], lw[2 * H:3 * H, 0], duw[0:H, 0],
                    diw[0:H, 0]], axis=1)
    a2 = jnp.stack([lw[H:2 * H, 0], lw[3 * H:4 * H, 0], duw[H:2 * H, 0],
                    diw[H:2 * H, 0]], axis=1)
    a1p = jnp.pad(a1, ((0, 0), (0, 124)))
    a2p = jnp.pad(a2, ((0, 0), (0, 124)))
    consts = jnp.pad(jnp.stack([lb, dub, dib], axis=0),
                     ((0, 5), (0, 0)))
    zer = jnp.zeros((ROWS_PT, CH), jnp.float32)

    sums = []
    for V, ei, ew, td in ((V_d1, edge_index_s, edge_weight_s, train_data_s),
                          (V_d2, edge_index_t, edge_weight_t, train_data_t)):
        vp = jnp.pad(V, ((0, NP - N), (0, 0)))
        src_t = jnp.pad(ei[0], (0, EPAD - E)).reshape(NT, NBK, KB)
        dst_t = jnp.pad(ei[1], (0, EPAD - E)).reshape(NT, NBK, KB)
        ew_t = jnp.pad(ew, (0, EPAD - E)).reshape(NT, NBK, KB)
        edata = jnp.stack(
            [src_t, dst_t, lax.bitcast_convert_type(ew_t, jnp.int32)], axis=2)
        u3 = td[:, 0].reshape(NB, 1, GB)
        i3 = td[:, 1].reshape(NB, 1, GB)
        y2 = td[:, 2].astype(jnp.float32).reshape(B, 1)

        sup1 = _mm(vp, W1)
        agg1 = _spmm(sup1, edata, zer)
        x1, sup2 = _mm2(agg1, b1r, W2)
        agg2 = _spmm(sup2, edata, zer)
        scores = _score(x1, agg2, b2r, a1p, a2p)
        gu, gi = _gather(u3, i3, scores)
        sums.append(_bce(gu, gi, y2, consts))

    s_s, s_t = sums[0][:, 0], sums[1][:, 0]
    loss = -(s_s[0] + s_t[0]) / B
    u_bce = -(s_s[2] + s_t[1]) / (2 * B)
    i_bce = -(s_s[4] + s_t[3]) / (2 * B)
    return loss + (i_bce + u_bce) * 0.1
